# Initial kernel scaffold; baseline (speedup 1.0000x reference)
import jax
import jax.numpy as jnp
from jax.experimental import pallas as pl


def kernel(x, edge_index, edge_attrs, batch, W1, b1, W2, b2, W3, b3, linW, linb):
    raise NotImplementedError("write your pallas kernel here")


# speedup vs baseline: 9.6441x; 9.6441x over previous
import functools

import jax
import jax.numpy as jnp
from jax import lax
from jax.experimental import pallas as pl
from jax.experimental.pallas import tpu as pltpu
from jax.experimental.pallas import tpu_sc as plsc

N_NODES = 10000
N_EDGES = 320000
FEAT = 128
N_GRAPHS = 64

NP = 10240
NW = 32
CHUNK = 128
NCHUNK = 81
N_ENT = NW * NCHUNK * CHUNK

@functools.lru_cache(maxsize=None)
def _make_deg_kernel():
    mesh = plsc.VectorSubcoreMesh(core_axis_name="c", subcore_axis_name="s")
    return pl.kernel(
        _deg_body,
        mesh=mesh,
        out_type=jax.ShapeDtypeStruct((2, NP), jnp.float32),
        scratch_types=[
            pltpu.VMEM((NCHUNK, CHUNK), jnp.int32),
            pltpu.VMEM((NCHUNK, CHUNK), jnp.float32),
            pltpu.VMEM((640,), jnp.float32),
            pltpu.VMEM_SHARED((NP,), jnp.float32),
        ],
    )


def _deg_body(col_hbm, ew_hbm, degp_hbm, colbuf, ewbuf, dloc, deg_sp):
    c = lax.axis_index("c")
    s = lax.axis_index("s")
    w = 2 * s + c

    pltpu.sync_copy(col_hbm.at[w], colbuf)
    pltpu.sync_copy(ew_hbm.at[w], ewbuf)

    def _z(i, _):
        dloc[pl.ds(i * 16, 16)] = jnp.zeros((16,), jnp.float32)
        return 0
    lax.fori_loop(0, 40, _z, 0)
    pltpu.sync_copy(dloc, deg_sp.at[pl.ds(s * 640, 640)])
    plsc.subcore_barrier()

    def _dg(g, _):
        pltpu.sync_copy(ewbuf.at[g], deg_sp.at[colbuf.at[g]], add=True)
        return 0
    lax.fori_loop(0, NCHUNK, _dg, 0)
    plsc.subcore_barrier()

    sl = pl.ds(s * 640, 640)
    pltpu.sync_copy(deg_sp.at[sl], degp_hbm.at[c, sl])


def _dis_body(p_ref, o_ref):
    o_ref[...] = lax.rsqrt(p_ref[0] + p_ref[1])


def _dis_tc(degp3d):
    return pl.pallas_call(
        _dis_body,
        out_shape=jax.ShapeDtypeStruct((NP // 128, 128), jnp.float32),
    )(degp3d)


@functools.lru_cache(maxsize=None)
def _make_norm_kernel():
    mesh = plsc.VectorSubcoreMesh(core_axis_name="c", subcore_axis_name="s")
    return pl.kernel(
        _norm_body,
        mesh=mesh,
        out_type=jax.ShapeDtypeStruct((NW, NCHUNK, CHUNK), jnp.float32),
        scratch_types=[
            pltpu.VMEM((NCHUNK, CHUNK), jnp.int32),
            pltpu.VMEM((NCHUNK, CHUNK), jnp.int32),
            pltpu.VMEM((NCHUNK, CHUNK), jnp.float32),
            pltpu.VMEM((CHUNK,), jnp.float32),
            pltpu.VMEM((CHUNK,), jnp.float32),
            pltpu.VMEM((CHUNK,), jnp.float32),
            pltpu.SemaphoreType.DMA,
            pltpu.SemaphoreType.DMA,
        ],
    )


def _norm_body(dis_hbm, row_hbm, col_hbm, ew_hbm, wts_hbm,
               rowbuf, colbuf, ewbuf, drbuf, dcbuf, nbuf, sem1, sem2):
    c = lax.axis_index("c")
    s = lax.axis_index("s")
    w = 2 * s + c

    pltpu.sync_copy(row_hbm.at[w], rowbuf)
    pltpu.sync_copy(col_hbm.at[w], colbuf)
    pltpu.sync_copy(ew_hbm.at[w], ewbuf)

    def _ng(g, _):
        cp1 = pltpu.async_copy(dis_hbm.at[rowbuf.at[g]], drbuf, sem1)
        cp2 = pltpu.async_copy(dis_hbm.at[colbuf.at[g]], dcbuf, sem2)
        cp1.wait()
        cp2.wait()

        def _ni(i, _2):
            ds16 = pl.ds(i * 16, 16)
            nbuf[ds16] = drbuf[ds16] * ewbuf[g, ds16] * dcbuf[ds16]
            return 0
        lax.fori_loop(0, 8, _ni, 0)
        pltpu.sync_copy(nbuf, wts_hbm.at[w, g])
        return 0
    lax.fori_loop(0, NCHUNK, _ng, 0)


@functools.lru_cache(maxsize=None)
def _make_edge_kernel():
    mesh = plsc.VectorSubcoreMesh(core_axis_name="c", subcore_axis_name="s")
    return pl.kernel(
        _edge_body,
        mesh=mesh,
        out_type=jax.ShapeDtypeStruct((2, NP, FEAT), jnp.float32),
        scratch_types=[
            pltpu.VMEM((NCHUNK, CHUNK), jnp.int32),
            pltpu.VMEM((NCHUNK, CHUNK), jnp.int32),
            pltpu.VMEM((CHUNK,), jnp.float32),
            pltpu.VMEM((CHUNK, FEAT), jnp.float32),
            pltpu.SemaphoreType.DMA,
            pltpu.VMEM_SHARED((NP, FEAT), jnp.float32),
        ],
    )


def _edge_body(h_hbm, row_hbm, col_hbm, wts_hbm, out_hbm,
               rowbuf, colbuf, wcbuf, gbuf, sem, acc_sp):
    c = lax.axis_index("c")
    s = lax.axis_index("s")
    w = 2 * s + c

    pltpu.sync_copy(row_hbm.at[w], rowbuf)
    pltpu.sync_copy(col_hbm.at[w], colbuf)

    def _z(i, _):
        for f in range(8):
            gbuf[i, pl.ds(f * 16, 16)] = jnp.zeros((16,), jnp.float32)
        return 0
    lax.fori_loop(0, CHUNK, _z, 0)
    for j in range(5):
        pltpu.sync_copy(gbuf, acc_sp.at[pl.ds(s * 640 + j * 128, 128)])
    plsc.subcore_barrier()

    def _eg(g, _):
        pltpu.sync_copy(wts_hbm.at[w, g], wcbuf)
        pltpu.async_copy(h_hbm.at[rowbuf.at[g]], gbuf, sem).wait()

        def _se(i, _2):
            wv = wcbuf[pl.ds(i * 16, 16)]
            for j in range(16):
                t = wv[j]
                e = i * 16 + j
                for f in range(8):
                    gbuf[e, pl.ds(f * 16, 16)] = gbuf[e, pl.ds(f * 16, 16)] * t
            return 0
        lax.fori_loop(0, CHUNK // 16, _se, 0)
        pltpu.sync_copy(gbuf, acc_sp.at[colbuf.at[g]], add=True)
        return 0
    lax.fori_loop(0, NCHUNK, _eg, 0)
    plsc.subcore_barrier()

    for j in range(5):
        sl = pl.ds(s * 640 + j * 128, 128)
        pltpu.sync_copy(acc_sp.at[sl], out_hbm.at[c, sl])


_BLK = 512
_NBLK = NP // _BLK


def _mm_body(x_ref, w_ref, o_ref):
    o_ref[...] = jnp.dot(x_ref[...], w_ref[...],
                         preferred_element_type=jnp.float32,
                         precision=lax.Precision.HIGHEST)


def _cmb_body(p_ref, b_ref, w_ref, o_ref):
    z = jax.nn.relu(p_ref[0] + p_ref[1] + b_ref[...])
    o_ref[...] = jnp.dot(z, w_ref[...],
                         preferred_element_type=jnp.float32,
                         precision=lax.Precision.HIGHEST)


def _pool_body(p_ref, b_ref, batch_ref, lw_ref, lb_ref, o_ref, psum, cnt):
    t = pl.program_id(0)

    @pl.when(t == 0)
    def _():
        psum[...] = jnp.zeros_like(psum)
        cnt[...] = jnp.zeros_like(cnt)

    z = p_ref[0] + p_ref[1] + b_ref[...]
    gcol = lax.broadcasted_iota(jnp.int32, (N_GRAPHS, 128), 0)
    for k in range(4):
        bk = batch_ref[0, k:k + 1, :]
        m = (gcol == bk).astype(jnp.float32)
        psum[...] += jnp.dot(m, z[k * 128:(k + 1) * 128, :],
                             preferred_element_type=jnp.float32,
                             precision=lax.Precision.HIGHEST)
        cnt[...] += jnp.sum(m, axis=1, keepdims=True)

    @pl.when(t == _NBLK - 1)
    def _():
        p = psum[...] / jnp.maximum(cnt[...], 1.0)
        o_ref[...] = jnp.dot(p, lw_ref[...],
                             preferred_element_type=jnp.float32,
                             precision=lax.Precision.HIGHEST) + lb_ref[...]


def _matmul(x_pad, W):
    return pl.pallas_call(
        _mm_body,
        grid=(_NBLK,),
        in_specs=[pl.BlockSpec((_BLK, FEAT), lambda t: (t, 0)),
                  pl.BlockSpec((FEAT, FEAT), lambda t: (0, 0))],
        out_specs=pl.BlockSpec((_BLK, FEAT), lambda t: (t, 0)),
        out_shape=jax.ShapeDtypeStruct((NP, FEAT), jnp.float32),
    )(x_pad, W)


def _combine_matmul(parts, b2d, W):
    return pl.pallas_call(
        _cmb_body,
        grid=(_NBLK,),
        in_specs=[pl.BlockSpec((2, _BLK, FEAT), lambda t: (0, t, 0)),
                  pl.BlockSpec((1, FEAT), lambda t: (0, 0)),
                  pl.BlockSpec((FEAT, FEAT), lambda t: (0, 0))],
        out_specs=pl.BlockSpec((_BLK, FEAT), lambda t: (t, 0)),
        out_shape=jax.ShapeDtypeStruct((NP, FEAT), jnp.float32),
    )(parts, b2d, W)


def _pool_head(parts, b2d, batch2d, lw_pad, lb2d):
    return pl.pallas_call(
        _pool_body,
        grid=(_NBLK,),
        in_specs=[pl.BlockSpec((2, _BLK, FEAT), lambda t: (0, t, 0)),
                  pl.BlockSpec((1, FEAT), lambda t: (0, 0)),
                  pl.BlockSpec((1, 4, 128), lambda t: (t, 0, 0)),
                  pl.BlockSpec((FEAT, 128), lambda t: (0, 0)),
                  pl.BlockSpec((1, 128), lambda t: (0, 0))],
        out_specs=pl.BlockSpec((N_GRAPHS, 128), lambda t: (0, 0)),
        out_shape=jax.ShapeDtypeStruct((N_GRAPHS, 128), jnp.float32),
        scratch_shapes=[pltpu.VMEM((N_GRAPHS, 128), jnp.float32),
                        pltpu.VMEM((N_GRAPHS, 128), jnp.float32)],
    )(parts, b2d, batch2d, lw_pad, lb2d)


def kernel(x, edge_index, edge_attrs, batch, W1, b1, W2, b2, W3, b3, linW, linb):
    f32 = jnp.float32
    row = edge_index[0]
    col = edge_index[1]
    loop = jnp.arange(NP, dtype=jnp.int32)
    pad_e = N_ENT - N_EDGES - NP
    zpad_i = jnp.zeros((pad_e,), jnp.int32)

    row_all = jnp.concatenate([row, loop, zpad_i]).reshape(NW, NCHUNK, CHUNK)
    col_all = jnp.concatenate([col, loop, zpad_i]).reshape(NW, NCHUNK, CHUNK)
    ew_all = jnp.concatenate(
        [edge_attrs, jnp.ones((NP,), f32), jnp.zeros((pad_e,), f32)]
    ).reshape(NW, NCHUNK, CHUNK)

    x_pad = jnp.concatenate([x, jnp.zeros((NP - N_NODES, FEAT), f32)])
    batch2d = jnp.concatenate(
        [batch, jnp.full((NP - N_NODES,), N_GRAPHS, jnp.int32)]
    ).reshape(_NBLK, 4, 128)
    b1_2d = b1.reshape(1, FEAT)
    b2_2d = b2.reshape(1, FEAT)
    b3_2d = b3.reshape(1, FEAT)
    lw_pad = jnp.pad(linW, ((0, 0), (0, 128 - linW.shape[1])))
    lb2d = jnp.broadcast_to(linb.reshape(1, 1), (1, 128))

    deg_kernel = _make_deg_kernel()
    norm_kernel = _make_norm_kernel()
    edge_kernel = _make_edge_kernel()

    degp = deg_kernel(col_all, ew_all)
    dis2d = _dis_tc(degp.reshape(2, NP // 128, 128))
    wts = norm_kernel(dis2d.reshape(NP), row_all, col_all, ew_all)

    h = _matmul(x_pad, W1)
    p = edge_kernel(h, row_all, col_all, wts)
    h = _combine_matmul(p, b1_2d, W2)
    p = edge_kernel(h, row_all, col_all, wts)
    h = _combine_matmul(p, b2_2d, W3)
    p = edge_kernel(h, row_all, col_all, wts)

    out128 = _pool_head(p, b3_2d, batch2d, lw_pad, lb2d)
    return out128[:, 0:1]


# speedup vs baseline: 12.1775x; 1.2627x over previous
import functools

import jax
import jax.numpy as jnp
from jax import lax
from jax.experimental import pallas as pl
from jax.experimental.pallas import tpu as pltpu
from jax.experimental.pallas import tpu_sc as plsc

N_NODES = 10000
N_EDGES = 320000
FEAT = 128
N_GRAPHS = 64

NP = 10240
NW = 32
CHUNK = 128
NCHUNK = 81
N_ENT = NW * NCHUNK * CHUNK

@functools.lru_cache(maxsize=None)
def _make_deg_kernel():
    mesh = plsc.VectorSubcoreMesh(core_axis_name="c", subcore_axis_name="s")
    return pl.kernel(
        _deg_body,
        mesh=mesh,
        out_type=jax.ShapeDtypeStruct((2, NP), jnp.float32),
        scratch_types=[
            pltpu.VMEM((NCHUNK, CHUNK), jnp.int32),
            pltpu.VMEM((NCHUNK, CHUNK), jnp.float32),
            pltpu.VMEM((640,), jnp.float32),
            pltpu.VMEM_SHARED((NP,), jnp.float32),
        ],
    )


def _deg_body(col_hbm, ew_hbm, degp_hbm, colbuf, ewbuf, dloc, deg_sp):
    c = lax.axis_index("c")
    s = lax.axis_index("s")
    w = 2 * s + c

    pltpu.sync_copy(col_hbm.at[w], colbuf)
    pltpu.sync_copy(ew_hbm.at[w], ewbuf)

    def _z(i, _):
        dloc[pl.ds(i * 16, 16)] = jnp.zeros((16,), jnp.float32)
        return 0
    lax.fori_loop(0, 40, _z, 0)
    pltpu.sync_copy(dloc, deg_sp.at[pl.ds(s * 640, 640)])
    plsc.subcore_barrier()

    def _dg(g, _):
        pltpu.sync_copy(ewbuf.at[g], deg_sp.at[colbuf.at[g]], add=True)
        return 0
    lax.fori_loop(0, NCHUNK, _dg, 0)
    plsc.subcore_barrier()

    sl = pl.ds(s * 640, 640)
    pltpu.sync_copy(deg_sp.at[sl], degp_hbm.at[c, sl])


def _dis_body(p_ref, o_ref):
    o_ref[...] = lax.rsqrt(p_ref[0] + p_ref[1])


def _dis_tc(degp3d):
    return pl.pallas_call(
        _dis_body,
        out_shape=jax.ShapeDtypeStruct((NP // 128, 128), jnp.float32),
    )(degp3d)


@functools.lru_cache(maxsize=None)
def _make_norm_kernel():
    mesh = plsc.VectorSubcoreMesh(core_axis_name="c", subcore_axis_name="s")
    return pl.kernel(
        _norm_body,
        mesh=mesh,
        out_type=jax.ShapeDtypeStruct((NW, NCHUNK, CHUNK), jnp.float32),
        scratch_types=[
            pltpu.VMEM((NCHUNK, CHUNK), jnp.int32),
            pltpu.VMEM((NCHUNK, CHUNK), jnp.int32),
            pltpu.VMEM((NCHUNK, CHUNK), jnp.float32),
            pltpu.VMEM((2, CHUNK), jnp.float32),
            pltpu.VMEM((2, CHUNK), jnp.float32),
            pltpu.VMEM((2, 1, CHUNK), jnp.float32),
            pltpu.SemaphoreType.DMA,
            pltpu.SemaphoreType.DMA,
            pltpu.SemaphoreType.DMA,
            pltpu.SemaphoreType.DMA,
            pltpu.SemaphoreType.DMA,
            pltpu.SemaphoreType.DMA,
        ],
    )


def _norm_body(dis_hbm, row_hbm, col_hbm, ew_hbm, wts_hbm,
               rowbuf, colbuf, ewbuf, drbuf, dcbuf, nbuf,
               ga0, gc0, ga1, gc1, st0, st1):
    c = lax.axis_index("c")
    s = lax.axis_index("s")
    w = 2 * s + c

    pltpu.sync_copy(row_hbm.at[w], rowbuf)
    pltpu.sync_copy(col_hbm.at[w], colbuf)
    pltpu.sync_copy(ew_hbm.at[w], ewbuf)

    def _gath(g, slot, sa, sc):
        pltpu.async_copy(dis_hbm.at[rowbuf.at[g]], drbuf.at[slot], sa)
        pltpu.async_copy(dis_hbm.at[colbuf.at[g]], dcbuf.at[slot], sc)

    def _gwait(g, slot, sa, sc):
        pltpu.make_async_copy(dis_hbm.at[rowbuf.at[g]], drbuf.at[slot], sa).wait()
        pltpu.make_async_copy(dis_hbm.at[colbuf.at[g]], dcbuf.at[slot], sc).wait()

    def _comp(g, slot):
        def _ni(i, _):
            ds16 = pl.ds(i * 16, 16)
            nbuf[slot, 0, ds16] = (drbuf[slot, ds16] * ewbuf[g, ds16]
                                   * dcbuf[slot, ds16])
            return 0
        lax.fori_loop(0, 8, _ni, 0)

    def _swait(g, slot, sem):
        pltpu.make_async_copy(nbuf.at[slot], wts_hbm.at[w, pl.ds(g, 1)],
                              sem).wait()

    NPAIR = NCHUNK // 2
    _gath(0, 0, ga0, gc0)

    def _it(k, _):
        a = 2 * k
        b = a + 1
        _gwait(a, 0, ga0, gc0)
        _gath(b, 1, ga1, gc1)

        @pl.when(k > 0)
        def _():
            _swait(a - 2, 0, st0)
        _comp(a, 0)
        _gath(a + 2, 0, ga0, gc0)
        pltpu.async_copy(nbuf.at[0], wts_hbm.at[w, pl.ds(a, 1)], st0)

        _gwait(b, 1, ga1, gc1)

        @pl.when(k > 0)
        def _():
            _swait(b - 2, 1, st1)
        _comp(b, 1)
        pltpu.async_copy(nbuf.at[1], wts_hbm.at[w, pl.ds(b, 1)], st1)
        return 0
    lax.fori_loop(0, NPAIR, _it, 0)

    g_last = NCHUNK - 1
    _gwait(g_last, 0, ga0, gc0)
    _swait(g_last - 2, 0, st0)
    _comp(g_last, 0)
    pltpu.async_copy(nbuf.at[0], wts_hbm.at[w, pl.ds(g_last, 1)], st0)
    _swait(g_last, 0, st0)
    _swait(g_last - 1, 1, st1)


ECHUNK = 64
ENCHUNK = N_ENT // (NW * ECHUNK)


@functools.lru_cache(maxsize=None)
def _make_edge_kernel():
    mesh = plsc.VectorSubcoreMesh(core_axis_name="c", subcore_axis_name="s")
    return pl.kernel(
        _edge_body,
        mesh=mesh,
        out_type=jax.ShapeDtypeStruct((2, NP, FEAT), jnp.float32),
        scratch_types=[
            pltpu.VMEM((NCHUNK, CHUNK), jnp.int32),
            pltpu.VMEM((NCHUNK, CHUNK), jnp.float32),
            pltpu.VMEM((2, ECHUNK), jnp.int32),
            pltpu.VMEM((2, ECHUNK), jnp.int32),
            pltpu.VMEM((ECHUNK, FEAT), jnp.float32),
            pltpu.VMEM((ECHUNK, FEAT), jnp.float32),
            pltpu.SemaphoreType.DMA,
            pltpu.SemaphoreType.DMA,
            pltpu.SemaphoreType.DMA,
            pltpu.SemaphoreType.DMA,
            pltpu.VMEM_SHARED((NP, FEAT), jnp.float32),
        ],
    )


def _edge_body(h_hbm, pk_hbm, wts_hbm, out_hbm,
               pbuf, wtsbuf, irow, icol, g0, g1, gs0, gs1, ss0, ss1, acc_sp):
    c = lax.axis_index("c")
    s = lax.axis_index("s")
    w = 2 * s + c

    pltpu.sync_copy(pk_hbm.at[w], pbuf)
    pltpu.sync_copy(wts_hbm.at[w], wtsbuf)

    def _unpack(g, slot):
        base = (g % 2) * ECHUNK
        for i in range(ECHUNK // 16):
            d16 = pl.ds(i * 16, 16)
            pv = pbuf[g // 2, pl.ds(base + i * 16, 16)]
            irow[slot, d16] = pv & jnp.int32(0xFFFF)
            icol[slot, d16] = lax.shift_right_logical(pv, 16)

    def _scale(gb, g):
        base = (g % 2) * ECHUNK

        def _se(i, _):
            wv = wtsbuf[g // 2, pl.ds(base + i * 16, 16)]
            for j in range(16):
                t = wv[j]
                e = i * 16 + j
                for f in range(8):
                    gb[e, pl.ds(f * 16, 16)] = gb[e, pl.ds(f * 16, 16)] * t
            return 0
        lax.fori_loop(0, ECHUNK // 16, _se, 0)

    def _z(i, _):
        for f in range(8):
            g0[i, pl.ds(f * 16, 16)] = jnp.zeros((16,), jnp.float32)
        return 0
    lax.fori_loop(0, ECHUNK, _z, 0)
    for j in range(10):
        pltpu.sync_copy(g0, acc_sp.at[pl.ds(s * 640 + j * ECHUNK, ECHUNK)])
    plsc.subcore_barrier()

    _unpack(0, 0)
    pltpu.async_copy(h_hbm.at[irow.at[0]], g0, gs0)

    def _it(k, _):
        a = 2 * k
        b = a + 1
        pltpu.make_async_copy(h_hbm.at[irow.at[0]], g0, gs0).wait()

        @pl.when(k > 0)
        def _():
            pltpu.make_async_copy(g1, acc_sp.at[icol.at[1]], ss1).wait()

        _unpack(b, 1)
        pltpu.async_copy(h_hbm.at[irow.at[1]], g1, gs1)
        _scale(g0, a)
        pltpu.async_copy(g0, acc_sp.at[icol.at[0]], ss0, add=True)

        pltpu.make_async_copy(h_hbm.at[irow.at[1]], g1, gs1).wait()
        _scale(g1, b)
        pltpu.make_async_copy(g0, acc_sp.at[icol.at[0]], ss0).wait()

        @pl.when(k < ENCHUNK // 2 - 1)
        def _():
            _unpack(a + 2, 0)
            pltpu.async_copy(h_hbm.at[irow.at[0]], g0, gs0)

        pltpu.async_copy(g1, acc_sp.at[icol.at[1]], ss1, add=True)
        return 0
    lax.fori_loop(0, ENCHUNK // 2, _it, 0)
    pltpu.make_async_copy(g1, acc_sp.at[icol.at[1]], ss1).wait()
    plsc.subcore_barrier()

    for j in range(5):
        sl = pl.ds(s * 640 + j * 128, 128)
        pltpu.sync_copy(acc_sp.at[sl], out_hbm.at[c, sl])


_BLK = 512
_NBLK = NP // _BLK


def _mm_body(x_ref, w_ref, o_ref):
    o_ref[...] = jnp.dot(x_ref[...], w_ref[...],
                         preferred_element_type=jnp.float32,
                         precision=lax.Precision.HIGHEST)


def _cmb_body(p_ref, b_ref, w_ref, o_ref):
    z = jax.nn.relu(p_ref[0] + p_ref[1] + b_ref[...])
    o_ref[...] = jnp.dot(z, w_ref[...],
                         preferred_element_type=jnp.float32,
                         precision=lax.Precision.HIGHEST)


def _pool_body(p_ref, b_ref, batch_ref, lw_ref, lb_ref, o_ref, psum, cnt):
    t = pl.program_id(0)

    @pl.when(t == 0)
    def _():
        psum[...] = jnp.zeros_like(psum)
        cnt[...] = jnp.zeros_like(cnt)

    z = p_ref[0] + p_ref[1] + b_ref[...]
    gcol = lax.broadcasted_iota(jnp.int32, (N_GRAPHS, 128), 0)
    for k in range(4):
        bk = batch_ref[0, k:k + 1, :]
        m = (gcol == bk).astype(jnp.float32)
        psum[...] += jnp.dot(m, z[k * 128:(k + 1) * 128, :],
                             preferred_element_type=jnp.float32,
                             precision=lax.Precision.HIGHEST)
        cnt[...] += jnp.sum(m, axis=1, keepdims=True)

    @pl.when(t == _NBLK - 1)
    def _():
        p = psum[...] / jnp.maximum(cnt[...], 1.0)
        o_ref[...] = jnp.dot(p, lw_ref[...],
                             preferred_element_type=jnp.float32,
                             precision=lax.Precision.HIGHEST) + lb_ref[...]


def _matmul(x_pad, W):
    return pl.pallas_call(
        _mm_body,
        grid=(_NBLK,),
        in_specs=[pl.BlockSpec((_BLK, FEAT), lambda t: (t, 0)),
                  pl.BlockSpec((FEAT, FEAT), lambda t: (0, 0))],
        out_specs=pl.BlockSpec((_BLK, FEAT), lambda t: (t, 0)),
        out_shape=jax.ShapeDtypeStruct((NP, FEAT), jnp.float32),
    )(x_pad, W)


def _combine_matmul(parts, b2d, W):
    return pl.pallas_call(
        _cmb_body,
        grid=(_NBLK,),
        in_specs=[pl.BlockSpec((2, _BLK, FEAT), lambda t: (0, t, 0)),
                  pl.BlockSpec((1, FEAT), lambda t: (0, 0)),
                  pl.BlockSpec((FEAT, FEAT), lambda t: (0, 0))],
        out_specs=pl.BlockSpec((_BLK, FEAT), lambda t: (t, 0)),
        out_shape=jax.ShapeDtypeStruct((NP, FEAT), jnp.float32),
    )(parts, b2d, W)


def _pool_head(parts, b2d, batch2d, lw_pad, lb2d):
    return pl.pallas_call(
        _pool_body,
        grid=(_NBLK,),
        in_specs=[pl.BlockSpec((2, _BLK, FEAT), lambda t: (0, t, 0)),
                  pl.BlockSpec((1, FEAT), lambda t: (0, 0)),
                  pl.BlockSpec((1, 4, 128), lambda t: (t, 0, 0)),
                  pl.BlockSpec((FEAT, 128), lambda t: (0, 0)),
                  pl.BlockSpec((1, 128), lambda t: (0, 0))],
        out_specs=pl.BlockSpec((N_GRAPHS, 128), lambda t: (0, 0)),
        out_shape=jax.ShapeDtypeStruct((N_GRAPHS, 128), jnp.float32),
        scratch_shapes=[pltpu.VMEM((N_GRAPHS, 128), jnp.float32),
                        pltpu.VMEM((N_GRAPHS, 128), jnp.float32)],
    )(parts, b2d, batch2d, lw_pad, lb2d)


def kernel(x, edge_index, edge_attrs, batch, W1, b1, W2, b2, W3, b3, linW, linb):
    f32 = jnp.float32
    row = edge_index[0]
    col = edge_index[1]
    loop = jnp.arange(NP, dtype=jnp.int32)
    pad_e = N_ENT - N_EDGES - NP
    zpad_i = jnp.zeros((pad_e,), jnp.int32)

    row_all = jnp.concatenate([row, loop, zpad_i]).reshape(NW, NCHUNK, CHUNK)
    col_all = jnp.concatenate([col, loop, zpad_i]).reshape(NW, NCHUNK, CHUNK)
    ew_all = jnp.concatenate(
        [edge_attrs, jnp.ones((NP,), f32), jnp.zeros((pad_e,), f32)]
    ).reshape(NW, NCHUNK, CHUNK)

    x_pad = jnp.concatenate([x, jnp.zeros((NP - N_NODES, FEAT), f32)])
    batch2d = jnp.concatenate(
        [batch, jnp.full((NP - N_NODES,), N_GRAPHS, jnp.int32)]
    ).reshape(_NBLK, 4, 128)
    b1_2d = b1.reshape(1, FEAT)
    b2_2d = b2.reshape(1, FEAT)
    b3_2d = b3.reshape(1, FEAT)
    lw_pad = jnp.pad(linW, ((0, 0), (0, 128 - linW.shape[1])))
    lb2d = jnp.broadcast_to(linb.reshape(1, 1), (1, 128))

    deg_kernel = _make_deg_kernel()
    norm_kernel = _make_norm_kernel()
    edge_kernel = _make_edge_kernel()

    degp = deg_kernel(col_all, ew_all)
    dis2d = _dis_tc(degp.reshape(2, NP // 128, 128))
    wts = norm_kernel(dis2d.reshape(NP), row_all, col_all, ew_all)

    packed = row_all + (col_all << 16)
    wts_e = wts

    h = _matmul(x_pad, W1)
    p = edge_kernel(h, packed, wts_e)
    h = _combine_matmul(p, b1_2d, W2)
    p = edge_kernel(h, packed, wts_e)
    h = _combine_matmul(p, b2_2d, W3)
    p = edge_kernel(h, packed, wts_e)

    out128 = _pool_head(p, b3_2d, batch2d, lw_pad, lb2d)
    return out128[:, 0:1]


# speedup vs baseline: 15.4220x; 1.2664x over previous
import functools

import jax
import jax.numpy as jnp
from jax import lax
from jax.experimental import pallas as pl
from jax.experimental.pallas import tpu as pltpu
from jax.experimental.pallas import tpu_sc as plsc

N_NODES = 10000
N_EDGES = 320000
FEAT = 128
N_GRAPHS = 64

NP = 10240
NW = 32
CHUNK = 128
NCHUNK = 81
N_ENT = NW * NCHUNK * CHUNK

@functools.lru_cache(maxsize=None)
def _make_deg_kernel():
    mesh = plsc.VectorSubcoreMesh(core_axis_name="c", subcore_axis_name="s")
    return pl.kernel(
        _deg_body,
        mesh=mesh,
        out_type=jax.ShapeDtypeStruct((2, NP), jnp.float32),
        scratch_types=[
            pltpu.VMEM((NCHUNK, CHUNK), jnp.int32),
            pltpu.VMEM((NCHUNK, CHUNK), jnp.float32),
            pltpu.VMEM((640,), jnp.float32),
            pltpu.VMEM_SHARED((NP,), jnp.float32),
        ],
    )


def _deg_body(col_hbm, ew_hbm, degp_hbm, colbuf, ewbuf, dloc, deg_sp):
    c = lax.axis_index("c")
    s = lax.axis_index("s")
    w = 2 * s + c

    pltpu.sync_copy(col_hbm.at[w], colbuf)
    pltpu.sync_copy(ew_hbm.at[w], ewbuf)

    def _z(i, _):
        dloc[pl.ds(i * 16, 16)] = jnp.zeros((16,), jnp.float32)
        return 0
    lax.fori_loop(0, 40, _z, 0)
    pltpu.sync_copy(dloc, deg_sp.at[pl.ds(s * 640, 640)])
    plsc.subcore_barrier()

    def _dg(g, _):
        pltpu.sync_copy(ewbuf.at[g], deg_sp.at[colbuf.at[g]], add=True)
        return 0
    lax.fori_loop(0, NCHUNK, _dg, 0)
    plsc.subcore_barrier()

    sl = pl.ds(s * 640, 640)
    pltpu.sync_copy(deg_sp.at[sl], degp_hbm.at[c, sl])


def _dis_body(p_ref, o_ref):
    o_ref[...] = lax.rsqrt(p_ref[0] + p_ref[1])


def _dis_tc(degp3d):
    return pl.pallas_call(
        _dis_body,
        out_shape=jax.ShapeDtypeStruct((NP // 128, 128), jnp.float32),
    )(degp3d)


@functools.lru_cache(maxsize=None)
def _make_norm_kernel():
    mesh = plsc.VectorSubcoreMesh(core_axis_name="c", subcore_axis_name="s")
    return pl.kernel(
        _norm_body,
        mesh=mesh,
        out_type=jax.ShapeDtypeStruct((NW, NCHUNK, CHUNK), jnp.float32),
        scratch_types=[
            pltpu.VMEM((NCHUNK, CHUNK), jnp.int32),
            pltpu.VMEM((NCHUNK, CHUNK), jnp.int32),
            pltpu.VMEM((NCHUNK, CHUNK), jnp.float32),
            pltpu.VMEM((2, CHUNK), jnp.float32),
            pltpu.VMEM((2, CHUNK), jnp.float32),
            pltpu.VMEM((2, 1, CHUNK), jnp.float32),
            pltpu.SemaphoreType.DMA,
            pltpu.SemaphoreType.DMA,
            pltpu.SemaphoreType.DMA,
            pltpu.SemaphoreType.DMA,
            pltpu.SemaphoreType.DMA,
            pltpu.SemaphoreType.DMA,
        ],
    )


def _norm_body(dis_hbm, row_hbm, col_hbm, ew_hbm, wts_hbm,
               rowbuf, colbuf, ewbuf, drbuf, dcbuf, nbuf,
               ga0, gc0, ga1, gc1, st0, st1):
    c = lax.axis_index("c")
    s = lax.axis_index("s")
    w = 2 * s + c

    pltpu.sync_copy(row_hbm.at[w], rowbuf)
    pltpu.sync_copy(col_hbm.at[w], colbuf)
    pltpu.sync_copy(ew_hbm.at[w], ewbuf)

    def _gath(g, slot, sa, sc):
        pltpu.async_copy(dis_hbm.at[rowbuf.at[g]], drbuf.at[slot], sa)
        pltpu.async_copy(dis_hbm.at[colbuf.at[g]], dcbuf.at[slot], sc)

    def _gwait(g, slot, sa, sc):
        pltpu.make_async_copy(dis_hbm.at[rowbuf.at[g]], drbuf.at[slot], sa).wait()
        pltpu.make_async_copy(dis_hbm.at[colbuf.at[g]], dcbuf.at[slot], sc).wait()

    def _comp(g, slot):
        def _ni(i, _):
            ds16 = pl.ds(i * 16, 16)
            nbuf[slot, 0, ds16] = (drbuf[slot, ds16] * ewbuf[g, ds16]
                                   * dcbuf[slot, ds16])
            return 0
        lax.fori_loop(0, 8, _ni, 0)

    def _swait(g, slot, sem):
        pltpu.make_async_copy(nbuf.at[slot], wts_hbm.at[w, pl.ds(g, 1)],
                              sem).wait()

    NPAIR = NCHUNK // 2
    _gath(0, 0, ga0, gc0)

    def _it(k, _):
        a = 2 * k
        b = a + 1
        _gwait(a, 0, ga0, gc0)
        _gath(b, 1, ga1, gc1)

        @pl.when(k > 0)
        def _():
            _swait(a - 2, 0, st0)
        _comp(a, 0)
        _gath(a + 2, 0, ga0, gc0)
        pltpu.async_copy(nbuf.at[0], wts_hbm.at[w, pl.ds(a, 1)], st0)

        _gwait(b, 1, ga1, gc1)

        @pl.when(k > 0)
        def _():
            _swait(b - 2, 1, st1)
        _comp(b, 1)
        pltpu.async_copy(nbuf.at[1], wts_hbm.at[w, pl.ds(b, 1)], st1)
        return 0
    lax.fori_loop(0, NPAIR, _it, 0)

    g_last = NCHUNK - 1
    _gwait(g_last, 0, ga0, gc0)
    _swait(g_last - 2, 0, st0)
    _comp(g_last, 0)
    pltpu.async_copy(nbuf.at[0], wts_hbm.at[w, pl.ds(g_last, 1)], st0)
    _swait(g_last, 0, st0)
    _swait(g_last - 1, 1, st1)


ECHUNK = 64
ENCHUNK = N_ENT // (NW * ECHUNK)


@functools.lru_cache(maxsize=None)
def _make_edge_kernel():
    mesh = plsc.VectorSubcoreMesh(core_axis_name="c", subcore_axis_name="s")
    return pl.kernel(
        _edge_body,
        mesh=mesh,
        out_type=jax.ShapeDtypeStruct((2, NP, FEAT), jnp.float32),
        scratch_types=[
            pltpu.VMEM((NCHUNK, CHUNK), jnp.int32),
            pltpu.VMEM((NCHUNK, CHUNK), jnp.float32),
            pltpu.VMEM((3, ECHUNK), jnp.int32),
            pltpu.VMEM((3, ECHUNK), jnp.int32),
            pltpu.VMEM((ECHUNK, FEAT), jnp.float32),
            pltpu.VMEM((ECHUNK, FEAT), jnp.float32),
            pltpu.VMEM((ECHUNK, FEAT), jnp.float32),
            pltpu.SemaphoreType.DMA,
            pltpu.SemaphoreType.DMA,
            pltpu.SemaphoreType.DMA,
            pltpu.SemaphoreType.DMA,
            pltpu.SemaphoreType.DMA,
            pltpu.SemaphoreType.DMA,
            pltpu.VMEM_SHARED((NP, FEAT), jnp.float32),
        ],
    )


def _edge_body(h_hbm, pk_hbm, wts_hbm, out_hbm,
               pbuf, wtsbuf, irow, icol, g0, g1, g2,
               gs0, gs1, gs2, ss0, ss1, ss2, acc_sp):
    c = lax.axis_index("c")
    s = lax.axis_index("s")
    w = 2 * s + c

    pltpu.sync_copy(pk_hbm.at[w], pbuf)
    pltpu.sync_copy(wts_hbm.at[w], wtsbuf)

    def _unpack(g, slot):
        base = (g % 2) * ECHUNK
        for i in range(ECHUNK // 16):
            d16 = pl.ds(i * 16, 16)
            pv = pbuf[g // 2, pl.ds(base + i * 16, 16)]
            irow[slot, d16] = pv & jnp.int32(0xFFFF)
            icol[slot, d16] = lax.shift_right_logical(pv, 16)

    def _scale(gb, g):
        base = (g % 2) * ECHUNK

        @plsc.parallel_loop(0, ECHUNK // 16, unroll=2)
        def _se(i):
            wv = wtsbuf[g // 2, pl.ds(base + i * 16, 16)]
            for j in range(16):
                t = wv[j]
                e = i * 16 + j
                for f in range(8):
                    gb[e, pl.ds(f * 16, 16)] = gb[e, pl.ds(f * 16, 16)] * t

    def _gstart(g, slot, gb, sem):
        _unpack(g, slot)
        pltpu.async_copy(h_hbm.at[irow.at[slot]], gb, sem)

    def _gwait(slot, gb, sem):
        pltpu.make_async_copy(h_hbm.at[irow.at[slot]], gb, sem).wait()

    def _sstart(slot, gb, sem):
        pltpu.async_copy(gb, acc_sp.at[icol.at[slot]], sem, add=True)

    def _swait(slot, gb, sem):
        pltpu.make_async_copy(gb, acc_sp.at[icol.at[slot]], sem).wait()

    def _z(i, _):
        for f in range(8):
            g0[i, pl.ds(f * 16, 16)] = jnp.zeros((16,), jnp.float32)
        return 0
    lax.fori_loop(0, ECHUNK, _z, 0)
    for j in range(10):
        pltpu.sync_copy(g0, acc_sp.at[pl.ds(s * 640 + j * ECHUNK, ECHUNK)])
    plsc.subcore_barrier()

    NTRI = ENCHUNK // 3
    _gstart(0, 0, g0, gs0)
    _gstart(1, 1, g1, gs1)

    def _it(k, _):
        a = 3 * k
        b = a + 1
        d = a + 2
        _gwait(0, g0, gs0)

        @pl.when(k > 0)
        def _():
            _swait(2, g2, ss2)
        _gstart(d, 2, g2, gs2)
        _scale(g0, a)
        _sstart(0, g0, ss0)

        _gwait(1, g1, gs1)
        _scale(g1, b)
        _sstart(1, g1, ss1)

        _swait(0, g0, ss0)

        @pl.when(k < NTRI - 1)
        def _():
            _gstart(a + 3, 0, g0, gs0)

        _gwait(2, g2, gs2)
        _scale(g2, d)
        _sstart(2, g2, ss2)

        _swait(1, g1, ss1)

        @pl.when(k < NTRI - 1)
        def _():
            _gstart(b + 3, 1, g1, gs1)
        return 0
    lax.fori_loop(0, NTRI, _it, 0)
    _swait(2, g2, ss2)
    plsc.subcore_barrier()

    for j in range(5):
        sl = pl.ds(s * 640 + j * 128, 128)
        pltpu.sync_copy(acc_sp.at[sl], out_hbm.at[c, sl])


_BLK = 512
_NBLK = NP // _BLK


def _mm_body(x_ref, w_ref, o_ref):
    o_ref[...] = jnp.dot(x_ref[...], w_ref[...],
                         preferred_element_type=jnp.float32,
                         precision=lax.Precision.HIGHEST)


def _cmb_body(p_ref, b_ref, w_ref, o_ref):
    z = jax.nn.relu(p_ref[0] + p_ref[1] + b_ref[...])
    o_ref[...] = jnp.dot(z, w_ref[...],
                         preferred_element_type=jnp.float32,
                         precision=lax.Precision.HIGHEST)


def _pool_body(p_ref, b_ref, batch_ref, lw_ref, lb_ref, o_ref, psum, cnt):
    t = pl.program_id(0)

    @pl.when(t == 0)
    def _():
        psum[...] = jnp.zeros_like(psum)
        cnt[...] = jnp.zeros_like(cnt)

    z = p_ref[0] + p_ref[1] + b_ref[...]
    gcol = lax.broadcasted_iota(jnp.int32, (N_GRAPHS, 128), 0)
    for k in range(4):
        bk = batch_ref[0, k:k + 1, :]
        m = (gcol == bk).astype(jnp.float32)
        psum[...] += jnp.dot(m, z[k * 128:(k + 1) * 128, :],
                             preferred_element_type=jnp.float32,
                             precision=lax.Precision.HIGHEST)
        cnt[...] += jnp.sum(m, axis=1, keepdims=True)

    @pl.when(t == _NBLK - 1)
    def _():
        p = psum[...] / jnp.maximum(cnt[...], 1.0)
        o_ref[...] = jnp.dot(p, lw_ref[...],
                             preferred_element_type=jnp.float32,
                             precision=lax.Precision.HIGHEST) + lb_ref[...]


def _matmul(x_pad, W):
    return pl.pallas_call(
        _mm_body,
        grid=(_NBLK,),
        in_specs=[pl.BlockSpec((_BLK, FEAT), lambda t: (t, 0)),
                  pl.BlockSpec((FEAT, FEAT), lambda t: (0, 0))],
        out_specs=pl.BlockSpec((_BLK, FEAT), lambda t: (t, 0)),
        out_shape=jax.ShapeDtypeStruct((NP, FEAT), jnp.float32),
    )(x_pad, W)


def _combine_matmul(parts, b2d, W):
    return pl.pallas_call(
        _cmb_body,
        grid=(_NBLK,),
        in_specs=[pl.BlockSpec((2, _BLK, FEAT), lambda t: (0, t, 0)),
                  pl.BlockSpec((1, FEAT), lambda t: (0, 0)),
                  pl.BlockSpec((FEAT, FEAT), lambda t: (0, 0))],
        out_specs=pl.BlockSpec((_BLK, FEAT), lambda t: (t, 0)),
        out_shape=jax.ShapeDtypeStruct((NP, FEAT), jnp.float32),
    )(parts, b2d, W)


def _pool_head(parts, b2d, batch2d, lw_pad, lb2d):
    return pl.pallas_call(
        _pool_body,
        grid=(_NBLK,),
        in_specs=[pl.BlockSpec((2, _BLK, FEAT), lambda t: (0, t, 0)),
                  pl.BlockSpec((1, FEAT), lambda t: (0, 0)),
                  pl.BlockSpec((1, 4, 128), lambda t: (t, 0, 0)),
                  pl.BlockSpec((FEAT, 128), lambda t: (0, 0)),
                  pl.BlockSpec((1, 128), lambda t: (0, 0))],
        out_specs=pl.BlockSpec((N_GRAPHS, 128), lambda t: (0, 0)),
        out_shape=jax.ShapeDtypeStruct((N_GRAPHS, 128), jnp.float32),
        scratch_shapes=[pltpu.VMEM((N_GRAPHS, 128), jnp.float32),
                        pltpu.VMEM((N_GRAPHS, 128), jnp.float32)],
    )(parts, b2d, batch2d, lw_pad, lb2d)


def kernel(x, edge_index, edge_attrs, batch, W1, b1, W2, b2, W3, b3, linW, linb):
    f32 = jnp.float32
    row = edge_index[0]
    col = edge_index[1]
    loop = jnp.arange(NP, dtype=jnp.int32)
    pad_e = N_ENT - N_EDGES - NP
    zpad_i = jnp.zeros((pad_e,), jnp.int32)

    row_all = jnp.concatenate([row, loop, zpad_i]).reshape(NW, NCHUNK, CHUNK)
    col_all = jnp.concatenate([col, loop, zpad_i]).reshape(NW, NCHUNK, CHUNK)
    ew_all = jnp.concatenate(
        [edge_attrs, jnp.ones((NP,), f32), jnp.zeros((pad_e,), f32)]
    ).reshape(NW, NCHUNK, CHUNK)

    x_pad = jnp.concatenate([x, jnp.zeros((NP - N_NODES, FEAT), f32)])
    batch2d = jnp.concatenate(
        [batch, jnp.full((NP - N_NODES,), N_GRAPHS, jnp.int32)]
    ).reshape(_NBLK, 4, 128)
    b1_2d = b1.reshape(1, FEAT)
    b2_2d = b2.reshape(1, FEAT)
    b3_2d = b3.reshape(1, FEAT)
    lw_pad = jnp.pad(linW, ((0, 0), (0, 128 - linW.shape[1])))
    lb2d = jnp.broadcast_to(linb.reshape(1, 1), (1, 128))

    deg_kernel = _make_deg_kernel()
    norm_kernel = _make_norm_kernel()
    edge_kernel = _make_edge_kernel()

    degp = deg_kernel(col_all, ew_all)
    dis2d = _dis_tc(degp.reshape(2, NP // 128, 128))
    wts = norm_kernel(dis2d.reshape(NP), row_all, col_all, ew_all)

    packed = row_all + (col_all << 16)
    wts_e = wts

    h = _matmul(x_pad, W1)
    p = edge_kernel(h, packed, wts_e)
    h = _combine_matmul(p, b1_2d, W2)
    p = edge_kernel(h, packed, wts_e)
    h = _combine_matmul(p, b2_2d, W3)
    p = edge_kernel(h, packed, wts_e)

    out128 = _pool_head(p, b3_2d, batch2d, lw_pad, lb2d)
    return out128[:, 0:1]


# speedup vs baseline: 17.0036x; 1.1026x over previous
import functools

import jax
import jax.numpy as jnp
from jax import lax
from jax.experimental import pallas as pl
from jax.experimental.pallas import tpu as pltpu
from jax.experimental.pallas import tpu_sc as plsc

N_NODES = 10000
N_EDGES = 320000
FEAT = 128
N_GRAPHS = 64

NP = 10240
NW = 32
CHUNK = 128
NCHUNK = 81
N_ENT = NW * NCHUNK * CHUNK

@functools.lru_cache(maxsize=None)
def _make_deg_kernel():
    mesh = plsc.VectorSubcoreMesh(core_axis_name="c", subcore_axis_name="s")
    return pl.kernel(
        _deg_body,
        mesh=mesh,
        out_type=jax.ShapeDtypeStruct((2, NP), jnp.float32),
        scratch_types=[
            pltpu.VMEM((NCHUNK, CHUNK), jnp.int32),
            pltpu.VMEM((NCHUNK, CHUNK), jnp.float32),
            pltpu.VMEM((640,), jnp.float32),
            pltpu.VMEM_SHARED((NP,), jnp.float32),
        ],
    )


def _deg_body(col_hbm, ew_hbm, degp_hbm, colbuf, ewbuf, dloc, deg_sp):
    c = lax.axis_index("c")
    s = lax.axis_index("s")
    w = 2 * s + c

    pltpu.sync_copy(col_hbm.at[w], colbuf)
    pltpu.sync_copy(ew_hbm.at[w], ewbuf)

    def _z(i, _):
        dloc[pl.ds(i * 16, 16)] = jnp.zeros((16,), jnp.float32)
        return 0
    lax.fori_loop(0, 40, _z, 0)
    pltpu.sync_copy(dloc, deg_sp.at[pl.ds(s * 640, 640)])
    plsc.subcore_barrier()

    def _dg(g, _):
        pltpu.sync_copy(ewbuf.at[g], deg_sp.at[colbuf.at[g]], add=True)
        return 0
    lax.fori_loop(0, NCHUNK, _dg, 0)
    plsc.subcore_barrier()

    sl = pl.ds(s * 640, 640)
    pltpu.sync_copy(deg_sp.at[sl], degp_hbm.at[c, sl])


def _dis_body(p_ref, o_ref):
    o_ref[...] = lax.rsqrt(p_ref[0] + p_ref[1])


def _dis_tc(degp3d):
    return pl.pallas_call(
        _dis_body,
        out_shape=jax.ShapeDtypeStruct((NP // 128, 128), jnp.float32),
    )(degp3d)


@functools.lru_cache(maxsize=None)
def _make_rep_kernel():
    mesh = plsc.VectorSubcoreMesh(core_axis_name="c", subcore_axis_name="s")
    return pl.kernel(
        _rep_body,
        mesh=mesh,
        out_type=jax.ShapeDtypeStruct((NP, FEAT), jnp.float32),
        scratch_types=[
            pltpu.VMEM((NP // NW,), jnp.float32),
            pltpu.VMEM((2, 16, FEAT), jnp.float32),
            pltpu.SemaphoreType.DMA,
            pltpu.SemaphoreType.DMA,
        ],
    )


def _rep_body(dis_hbm, rep_hbm, dloc, rbuf, st0, st1):
    c = lax.axis_index("c")
    s = lax.axis_index("s")
    w = 2 * s + c
    npw = NP // NW
    base = w * npw

    pltpu.sync_copy(dis_hbm.at[pl.ds(base, npw)], dloc)

    def _fill(grp, slot):
        dv = dloc[pl.ds(grp * 16, 16)]
        for j in range(16):
            t = dv[j]
            for f in range(8):
                rbuf[slot, j, pl.ds(f * 16, 16)] = jnp.broadcast_to(t, (16,))

    def _store(grp, slot, sem):
        pltpu.async_copy(rbuf.at[slot],
                         rep_hbm.at[pl.ds(base + grp * 16, 16)], sem)

    def _swait(grp, slot, sem):
        pltpu.make_async_copy(rbuf.at[slot],
                              rep_hbm.at[pl.ds(base + grp * 16, 16)],
                              sem).wait()

    NG = npw // 16

    def _it(k, _):
        a = 2 * k
        b = a + 1

        @pl.when(k > 0)
        def _():
            _swait(a - 2, 0, st0)
        _fill(a, 0)
        _store(a, 0, st0)

        @pl.when(k > 0)
        def _():
            _swait(b - 2, 1, st1)
        _fill(b, 1)
        _store(b, 1, st1)
        return 0
    lax.fori_loop(0, NG // 2, _it, 0)
    _swait(NG - 2, 0, st0)
    _swait(NG - 1, 1, st1)


ECHUNK = 64
ENCHUNK = N_ENT // (NW * ECHUNK)


@functools.lru_cache(maxsize=None)
def _make_edge_kernel():
    mesh = plsc.VectorSubcoreMesh(core_axis_name="c", subcore_axis_name="s")
    return pl.kernel(
        _edge_body,
        mesh=mesh,
        out_type=jax.ShapeDtypeStruct((2, NP, FEAT), jnp.float32),
        scratch_types=[
            pltpu.VMEM((NCHUNK, CHUNK), jnp.int32),
            pltpu.VMEM((NCHUNK, CHUNK), jnp.float32),
            pltpu.VMEM((3, ECHUNK), jnp.int32),
            pltpu.VMEM((3, ECHUNK), jnp.int32),
            pltpu.VMEM((ECHUNK, FEAT), jnp.float32),
            pltpu.VMEM((ECHUNK, FEAT), jnp.float32),
            pltpu.VMEM((ECHUNK, FEAT), jnp.float32),
            pltpu.SemaphoreType.DMA,
            pltpu.SemaphoreType.DMA,
            pltpu.SemaphoreType.DMA,
            pltpu.SemaphoreType.DMA,
            pltpu.SemaphoreType.DMA,
            pltpu.SemaphoreType.DMA,
            pltpu.VMEM_SHARED((NP, FEAT), jnp.float32),
        ],
    )


def _edge_body(h_hbm, pk_hbm, wts_hbm, out_hbm,
               pbuf, wtsbuf, irow, icol, g0, g1, g2,
               gs0, gs1, gs2, ss0, ss1, ss2, acc_sp):
    c = lax.axis_index("c")
    s = lax.axis_index("s")
    w = 2 * s + c

    pltpu.sync_copy(pk_hbm.at[w], pbuf)
    pltpu.sync_copy(wts_hbm.at[w], wtsbuf)

    def _unpack(g, slot):
        base = (g % 2) * ECHUNK
        for i in range(ECHUNK // 16):
            d16 = pl.ds(i * 16, 16)
            pv = pbuf[g // 2, pl.ds(base + i * 16, 16)]
            irow[slot, d16] = pv & jnp.int32(0xFFFF)
            icol[slot, d16] = lax.shift_right_logical(pv, 16)

    def _scale(gb, g):
        base = (g % 2) * ECHUNK

        @plsc.parallel_loop(0, ECHUNK // 16, unroll=2)
        def _se(i):
            wv = wtsbuf[g // 2, pl.ds(base + i * 16, 16)]
            for j in range(16):
                t = wv[j]
                e = i * 16 + j
                for f in range(8):
                    gb[e, pl.ds(f * 16, 16)] = gb[e, pl.ds(f * 16, 16)] * t

    def _gstart(g, slot, gb, sem):
        _unpack(g, slot)
        pltpu.async_copy(h_hbm.at[irow.at[slot]], gb, sem)

    def _gwait(slot, gb, sem):
        pltpu.make_async_copy(h_hbm.at[irow.at[slot]], gb, sem).wait()

    def _sstart(slot, gb, sem):
        pltpu.async_copy(gb, acc_sp.at[icol.at[slot]], sem, add=True)

    def _swait(slot, gb, sem):
        pltpu.make_async_copy(gb, acc_sp.at[icol.at[slot]], sem).wait()

    def _z(i, _):
        for f in range(8):
            g0[i, pl.ds(f * 16, 16)] = jnp.zeros((16,), jnp.float32)
        return 0
    lax.fori_loop(0, ECHUNK, _z, 0)
    for j in range(10):
        pltpu.sync_copy(g0, acc_sp.at[pl.ds(s * 640 + j * ECHUNK, ECHUNK)])
    plsc.subcore_barrier()

    NTRI = ENCHUNK // 3
    _gstart(0, 0, g0, gs0)
    _gstart(1, 1, g1, gs1)

    def _it(k, _):
        a = 3 * k
        b = a + 1
        d = a + 2
        _gwait(0, g0, gs0)

        @pl.when(k > 0)
        def _():
            _swait(2, g2, ss2)
        _gstart(d, 2, g2, gs2)
        _scale(g0, a)
        _sstart(0, g0, ss0)

        _gwait(1, g1, gs1)
        _scale(g1, b)
        _sstart(1, g1, ss1)

        _swait(0, g0, ss0)

        @pl.when(k < NTRI - 1)
        def _():
            _gstart(a + 3, 0, g0, gs0)

        _gwait(2, g2, gs2)
        _scale(g2, d)
        _sstart(2, g2, ss2)

        _swait(1, g1, ss1)

        @pl.when(k < NTRI - 1)
        def _():
            _gstart(b + 3, 1, g1, gs1)
        return 0
    lax.fori_loop(0, NTRI, _it, 0)
    _swait(2, g2, ss2)
    plsc.subcore_barrier()

    for j in range(5):
        sl = pl.ds(s * 640 + j * 128, 128)
        pltpu.sync_copy(acc_sp.at[sl], out_hbm.at[c, sl])


_BLK = 512
_NBLK = NP // _BLK


def _mm_body(x_ref, d_ref, w_ref, o_ref):
    o_ref[...] = d_ref[...] * jnp.dot(x_ref[...], w_ref[...],
                                      preferred_element_type=jnp.float32,
                                      precision=lax.Precision.HIGHEST)


def _cmb_body(p_ref, d_ref, b_ref, w_ref, o_ref):
    z = jax.nn.relu(d_ref[...] * (p_ref[0] + p_ref[1]) + b_ref[...])
    o_ref[...] = d_ref[...] * jnp.dot(z, w_ref[...],
                                      preferred_element_type=jnp.float32,
                                      precision=lax.Precision.HIGHEST)


def _pool_body(p_ref, d_ref, b_ref, batch_ref, lw_ref, lb_ref, o_ref, psum, cnt):
    t = pl.program_id(0)

    @pl.when(t == 0)
    def _():
        psum[...] = jnp.zeros_like(psum)
        cnt[...] = jnp.zeros_like(cnt)

    z = d_ref[...] * (p_ref[0] + p_ref[1]) + b_ref[...]
    gcol = lax.broadcasted_iota(jnp.int32, (N_GRAPHS, 128), 0)
    for k in range(4):
        bk = batch_ref[0, k:k + 1, :]
        m = (gcol == bk).astype(jnp.float32)
        psum[...] += jnp.dot(m, z[k * 128:(k + 1) * 128, :],
                             preferred_element_type=jnp.float32,
                             precision=lax.Precision.HIGHEST)
        cnt[...] += jnp.sum(m, axis=1, keepdims=True)

    @pl.when(t == _NBLK - 1)
    def _():
        p = psum[...] / jnp.maximum(cnt[...], 1.0)
        o_ref[...] = jnp.dot(p, lw_ref[...],
                             preferred_element_type=jnp.float32,
                             precision=lax.Precision.HIGHEST) + lb_ref[...]


def _matmul(x_pad, dis_rep, W):
    return pl.pallas_call(
        _mm_body,
        grid=(_NBLK,),
        in_specs=[pl.BlockSpec((_BLK, FEAT), lambda t: (t, 0)),
                  pl.BlockSpec((_BLK, FEAT), lambda t: (t, 0)),
                  pl.BlockSpec((FEAT, FEAT), lambda t: (0, 0))],
        out_specs=pl.BlockSpec((_BLK, FEAT), lambda t: (t, 0)),
        out_shape=jax.ShapeDtypeStruct((NP, FEAT), jnp.float32),
    )(x_pad, dis_rep, W)


def _combine_matmul(parts, dis_rep, b2d, W):
    return pl.pallas_call(
        _cmb_body,
        grid=(_NBLK,),
        in_specs=[pl.BlockSpec((2, _BLK, FEAT), lambda t: (0, t, 0)),
                  pl.BlockSpec((_BLK, FEAT), lambda t: (t, 0)),
                  pl.BlockSpec((1, FEAT), lambda t: (0, 0)),
                  pl.BlockSpec((FEAT, FEAT), lambda t: (0, 0))],
        out_specs=pl.BlockSpec((_BLK, FEAT), lambda t: (t, 0)),
        out_shape=jax.ShapeDtypeStruct((NP, FEAT), jnp.float32),
    )(parts, dis_rep, b2d, W)


def _pool_head(parts, dis_rep, b2d, batch2d, lw_pad, lb2d):
    return pl.pallas_call(
        _pool_body,
        grid=(_NBLK,),
        in_specs=[pl.BlockSpec((2, _BLK, FEAT), lambda t: (0, t, 0)),
                  pl.BlockSpec((_BLK, FEAT), lambda t: (t, 0)),
                  pl.BlockSpec((1, FEAT), lambda t: (0, 0)),
                  pl.BlockSpec((1, 4, 128), lambda t: (t, 0, 0)),
                  pl.BlockSpec((FEAT, 128), lambda t: (0, 0)),
                  pl.BlockSpec((1, 128), lambda t: (0, 0))],
        out_specs=pl.BlockSpec((N_GRAPHS, 128), lambda t: (0, 0)),
        out_shape=jax.ShapeDtypeStruct((N_GRAPHS, 128), jnp.float32),
        scratch_shapes=[pltpu.VMEM((N_GRAPHS, 128), jnp.float32),
                        pltpu.VMEM((N_GRAPHS, 128), jnp.float32)],
    )(parts, dis_rep, b2d, batch2d, lw_pad, lb2d)


def kernel(x, edge_index, edge_attrs, batch, W1, b1, W2, b2, W3, b3, linW, linb):
    f32 = jnp.float32
    row = edge_index[0]
    col = edge_index[1]
    loop = jnp.arange(NP, dtype=jnp.int32)
    pad_e = N_ENT - N_EDGES - NP
    zpad_i = jnp.zeros((pad_e,), jnp.int32)

    row_all = jnp.concatenate([row, loop, zpad_i]).reshape(NW, NCHUNK, CHUNK)
    col_all = jnp.concatenate([col, loop, zpad_i]).reshape(NW, NCHUNK, CHUNK)
    ew_all = jnp.concatenate(
        [edge_attrs, jnp.ones((NP,), f32), jnp.zeros((pad_e,), f32)]
    ).reshape(NW, NCHUNK, CHUNK)

    x_pad = jnp.concatenate([x, jnp.zeros((NP - N_NODES, FEAT), f32)])
    batch2d = jnp.concatenate(
        [batch, jnp.full((NP - N_NODES,), N_GRAPHS, jnp.int32)]
    ).reshape(_NBLK, 4, 128)
    b1_2d = b1.reshape(1, FEAT)
    b2_2d = b2.reshape(1, FEAT)
    b3_2d = b3.reshape(1, FEAT)
    lw_pad = jnp.pad(linW, ((0, 0), (0, 128 - linW.shape[1])))
    lb2d = jnp.broadcast_to(linb.reshape(1, 1), (1, 128))

    deg_kernel = _make_deg_kernel()
    rep_kernel = _make_rep_kernel()
    edge_kernel = _make_edge_kernel()

    degp = deg_kernel(col_all, ew_all)
    dis2d = _dis_tc(degp.reshape(2, NP // 128, 128))
    dis_rep = rep_kernel(dis2d.reshape(NP))

    packed = row_all + (col_all << 16)

    h = _matmul(x_pad, dis_rep, W1)
    p = edge_kernel(h, packed, ew_all)
    h = _combine_matmul(p, dis_rep, b1_2d, W2)
    p = edge_kernel(h, packed, ew_all)
    h = _combine_matmul(p, dis_rep, b2_2d, W3)
    p = edge_kernel(h, packed, ew_all)

    out128 = _pool_head(p, dis_rep, b3_2d, batch2d, lw_pad, lb2d)
    return out128[:, 0:1]
